# baseline (speedup 1.0000x reference)
import functools

import jax
import jax.numpy as jnp
from jax import lax
from jax.experimental import pallas as pl
from jax.experimental.pallas import tpu as pltpu
from jax.experimental.pallas import tpu_sc as plsc

D = 128
HEADS = 8
HD = D // HEADS
B = 2048
FAN1 = 16
FAN2 = 16

N1 = B * FAN1
E2 = N1 * FAN2
CHUNK_SEEDS = (128, 640, 768, 512)
NCHK = len(CHUNK_SEEDS)


_NC = 2
_NS = 16
_NW = _NC * _NS

_CH2 = 128

_R1 = N1 + B
_RPW1 = _R1 // _NW
_CH1 = 64
_NCH1 = _RPW1 // _CH1


def _gather_loop(tab_hbm, idx_all, out_hbm, base, ch, nch, rows0, rows1, sem0, sem1):
    def _fire(g, rows, sem):
        off = pl.multiple_of(g * ch, 8)
        pltpu.make_async_copy(
            tab_hbm.at[idx_all.at[pl.ds(off, ch)]], rows, sem).start()

    def _drain(g, rows, sem):
        off = pl.multiple_of(g * ch, 8)
        pltpu.make_async_copy(
            tab_hbm.at[idx_all.at[pl.ds(off, ch)]], rows, sem).wait()
        pltpu.sync_copy(rows, out_hbm.at[pl.ds(pl.multiple_of(base + off, 8), ch)])

    _fire(0, rows0, sem0)

    def _pair(t, carry):
        a = 2 * t
        b = a + 1
        _fire(b, rows1, sem1)
        _drain(a, rows0, sem0)

        @pl.when(b + 1 < nch)
        def _():
            _fire(b + 1, rows0, sem0)

        _drain(b, rows1, sem1)
        return carry

    lax.fori_loop(0, nch // 2, _pair, 0)
    if nch % 2:
        _drain(nch - 1, rows0, sem0)


_SEMS = [pltpu.SemaphoreType.DMA, pltpu.SemaphoreType.DMA]


@functools.cache
def _sc_edge_fn(ec):
    rpw = ec // _NW
    nch = rpw // _CH2

    def body(emb_hbm, idx2_hbm, out2_hbm, idx2_all, r2a, r2b, sem0, sem1):
        wid = lax.axis_index("s") * _NC + lax.axis_index("c")
        base2 = pl.multiple_of(wid * rpw, 8)
        pltpu.sync_copy(idx2_hbm.at[pl.ds(base2, rpw)], idx2_all)
        _gather_loop(emb_hbm, idx2_all, out2_hbm, base2, _CH2, nch,
                     r2a, r2b, sem0, sem1)

    return pl.kernel(
        body,
        out_type=jax.ShapeDtypeStruct((ec, D), jnp.float32),
        mesh=plsc.VectorSubcoreMesh(core_axis_name="c", subcore_axis_name="s"),
        scratch_types=[
            pltpu.VMEM((rpw,), jnp.int32),
            pltpu.VMEM((_CH2, D), jnp.float32),
            pltpu.VMEM((_CH2, D), jnp.float32),
        ] + _SEMS,
    )


def _sc_gather_self_body(emb_hbm, idx1_hbm, out1_hbm,
                         idx1_all, r1a, r1b, sem0, sem1):
    wid = lax.axis_index("s") * _NC + lax.axis_index("c")
    base1 = pl.multiple_of(wid * _RPW1, 8)
    pltpu.sync_copy(idx1_hbm.at[pl.ds(base1, _RPW1)], idx1_all)
    _gather_loop(emb_hbm, idx1_all, out1_hbm, base1, _CH1, _NCH1,
                 r1a, r1b, sem0, sem1)


@functools.cache
def _sc_self_fn():
    return pl.kernel(
        _sc_gather_self_body,
        out_type=jax.ShapeDtypeStruct((_R1, D), jnp.float32),
        mesh=plsc.VectorSubcoreMesh(core_axis_name="c", subcore_axis_name="s"),
        scratch_types=[
            pltpu.VMEM((_RPW1,), jnp.int32),
            pltpu.VMEM((_CH1, D), jnp.float32),
            pltpu.VMEM((_CH1, D), jnp.float32),
        ] + _SEMS,
    )


BS = 16
BN = BS * FAN1


def _matT(a, w):
    return lax.dot_general(a, w, (((1,), (1,)), ((), ())),
                           preferred_element_type=jnp.float32)


def _tc_gat_body(h2_ref, hs_ref, h0_ref, wq1, wk1, wv1, ws1,
                 wq2, wk2, wv2, ws2, out_ref):
    f32 = jnp.float32
    scale = float(HD) ** (-0.5)
    SS = (lax.broadcasted_iota(jnp.int32, (D, D), 0) // HD
          == lax.broadcasted_iota(jnp.int32, (D, D), 1) // HD).astype(jnp.bfloat16)

    h2 = h2_ref[...].reshape(FAN2 * BN, D)
    hs = hs_ref[...]
    h0 = h0_ref[...]

    def _gat(h_self, h_neigh, fanout, wq, wk, wv, ws, kv_bf16=False,
             fan_major=False):
        n = h_self.shape[0]
        q = _matT(h_self, wq) * scale
        if kv_bf16:
            hn = h_neigh.astype(jnp.bfloat16)
            k = _matT(hn, wk.astype(jnp.bfloat16))
            v = _matT(hn, wv.astype(jnp.bfloat16))
        else:
            k = _matT(h_neigh, wk)
            v = _matT(h_neigh, wv)
        if fan_major:
            k3 = k.reshape(fanout, n, D)
            kq = (k3 * q[None, :, :]).reshape(fanout * n, D)
        else:
            k3 = k.reshape(n, fanout, D)
            kq = (k3 * q[:, None, :]).reshape(n * fanout, D)
        scf = lax.dot_general(kq.astype(jnp.bfloat16), SS,
                              (((1,), (0,)), ((), ())),
                              preferred_element_type=f32)
        if fan_major:
            p = jnp.exp(scf.reshape(fanout, n, D))
            s = jnp.sum(p, axis=0)
            agg = jnp.sum(p * v.reshape(fanout, n, D), axis=0) / s
        else:
            p = jnp.exp(scf.reshape(n, fanout, D))
            s = jnp.sum(p, axis=1)
            agg = jnp.sum(p * v.reshape(n, fanout, D), axis=1) / s
        return _matT(h_self, ws) + agg

    h1 = jnp.maximum(_gat(hs, h2, FAN2, wq1[...], wk1[...], wv1[...], ws1[...],
                          kv_bf16=True, fan_major=True), 0.0)
    out = jnp.maximum(_gat(h0, h1, FAN1, wq2[...], wk2[...], wv2[...], ws2[...],
                           kv_bf16=True), 0.0)
    out_ref[...] = out


def _tc_gat_chunk(seed_off, nseeds, rows2c, rows1,
                  Wq1, Wk1, Wv1, Ws1, Wq2, Wk2, Wv2, Ws2, interpret=False):
    wspec = pl.BlockSpec((D, D), lambda j: (0, 0))
    gridc = nseeds // BS
    off1 = seed_off // BS
    off0 = N1 // BS + seed_off // BS
    return pl.pallas_call(
        _tc_gat_body,
        grid=(gridc,),
        in_specs=[
            pl.BlockSpec((FAN2, BN, D), lambda j: (0, j, 0)),
            pl.BlockSpec((BN, D), lambda j: (j + off1, 0)),
            pl.BlockSpec((BS, D), lambda j: (j + off0, 0)),
            wspec, wspec, wspec, wspec, wspec, wspec, wspec, wspec,
        ],
        out_specs=pl.BlockSpec((BS, D), lambda j: (j, 0)),
        out_shape=jax.ShapeDtypeStruct((nseeds, D), jnp.float32),
        interpret=interpret,
    )(rows2c, rows1, rows1, Wq1, Wk1, Wv1, Ws1, Wq2, Wk2, Wv2, Ws2)


def kernel(seeds, nbr1, nbr2, emb, Wq1, Wk1, Wv1, Ws1, Wq2, Wk2, Wv2, Ws2):
    idx1 = jnp.concatenate([nbr1, seeds]).astype(jnp.int32)
    weights = (Wq1, Wk1, Wv1, Ws1, Wq2, Wk2, Wv2, Ws2)
    rows1 = _sc_self_fn()(emb, idx1)
    outs = []
    seed_off = 0
    for ns in CHUNK_SEEDS:
        npc = ns * FAN1
        n0 = seed_off * FAN1
        idx2c = nbr2[n0:n0 + npc, :].T.reshape(-1).astype(jnp.int32)
        rows2c = _sc_edge_fn(npc * FAN2)(emb, idx2c)
        outs.append(_tc_gat_chunk(seed_off, ns, rows2c.reshape(FAN2, npc, D),
                                  rows1, *weights))
        seed_off += ns
    return jnp.concatenate(outs, axis=0)


# speedup vs baseline: 1.0181x; 1.0181x over previous
import functools

import jax
import jax.numpy as jnp
from jax import lax
from jax.experimental import pallas as pl
from jax.experimental.pallas import tpu as pltpu
from jax.experimental.pallas import tpu_sc as plsc

D = 128
HEADS = 8
HD = D // HEADS
B = 2048
FAN1 = 16
FAN2 = 16

N1 = B * FAN1
E2 = N1 * FAN2
CHUNK_SEEDS = (512, 512, 512, 512)
NCHK = len(CHUNK_SEEDS)


_NC = 2
_NS = 16
_NW = _NC * _NS

_CH2 = 128

_R1 = N1 + B
_RPW1 = _R1 // _NW
_CH1 = 64
_NCH1 = _RPW1 // _CH1


_NB = 4


def _gather_loop(tab_hbm, idx_all, out_hbm, base, ch, nch, bufs, gsems, wsems):
    def _gfire(g, s):
        off = pl.multiple_of(g * ch, 8)
        pltpu.make_async_copy(
            tab_hbm.at[idx_all.at[pl.ds(off, ch)]], bufs[s], gsems[s]).start()

    def _gwait(g, s):
        off = pl.multiple_of(g * ch, 8)
        pltpu.make_async_copy(
            tab_hbm.at[idx_all.at[pl.ds(off, ch)]], bufs[s], gsems[s]).wait()

    def _wfire(g, s):
        off = pl.multiple_of(base + g * ch, 8)
        pltpu.make_async_copy(
            bufs[s], out_hbm.at[pl.ds(off, ch)], wsems[s]).start()

    def _wwait(s):
        pltpu.make_async_copy(
            bufs[s], out_hbm.at[pl.ds(base, ch)], wsems[s]).wait()

    ng, rem = nch // _NB, nch % _NB

    def _group(t, carry):
        for s in range(_NB):
            @pl.when(t > 0)
            def _():
                _wwait(s)
            _gfire(_NB * t + s, s)
        for s in range(_NB):
            _gwait(_NB * t + s, s)
            _wfire(_NB * t + s, s)
        return carry

    lax.fori_loop(0, ng, _group, 0)
    for s in range(rem):
        if ng > 0:
            _wwait(s)
        _gfire(ng * _NB + s, s)
    for s in range(rem):
        _gwait(ng * _NB + s, s)
        _wfire(ng * _NB + s, s)
    for s in range(_NB if ng > 0 else rem):
        _wwait(s)


_SEMS = [pltpu.SemaphoreType.DMA] * (2 * _NB)


@functools.cache
def _sc_edge_fn(ec):
    rpw = ec // _NW
    nch = rpw // _CH2

    def body(emb_hbm, idx2_hbm, out2_hbm, idx2_all, *rest):
        bufs, sems = rest[:_NB], rest[_NB:]
        wid = lax.axis_index("s") * _NC + lax.axis_index("c")
        base2 = pl.multiple_of(wid * rpw, 8)
        pltpu.sync_copy(idx2_hbm.at[pl.ds(base2, rpw)], idx2_all)
        _gather_loop(emb_hbm, idx2_all, out2_hbm, base2, _CH2, nch,
                     bufs, sems[:_NB], sems[_NB:])

    return pl.kernel(
        body,
        out_type=jax.ShapeDtypeStruct((ec, D), jnp.float32),
        mesh=plsc.VectorSubcoreMesh(core_axis_name="c", subcore_axis_name="s"),
        scratch_types=[pltpu.VMEM((rpw,), jnp.int32)]
        + [pltpu.VMEM((_CH2, D), jnp.float32)] * _NB + _SEMS,
    )


def _sc_gather_self_body(emb_hbm, idx1_hbm, out1_hbm, idx1_all, *rest):
    bufs, sems = rest[:_NB], rest[_NB:]
    wid = lax.axis_index("s") * _NC + lax.axis_index("c")
    base1 = pl.multiple_of(wid * _RPW1, 8)
    pltpu.sync_copy(idx1_hbm.at[pl.ds(base1, _RPW1)], idx1_all)
    _gather_loop(emb_hbm, idx1_all, out1_hbm, base1, _CH1, _NCH1,
                 bufs, sems[:_NB], sems[_NB:])


@functools.cache
def _sc_self_fn():
    return pl.kernel(
        _sc_gather_self_body,
        out_type=jax.ShapeDtypeStruct((_R1, D), jnp.float32),
        mesh=plsc.VectorSubcoreMesh(core_axis_name="c", subcore_axis_name="s"),
        scratch_types=[pltpu.VMEM((_RPW1,), jnp.int32)]
        + [pltpu.VMEM((_CH1, D), jnp.float32)] * _NB + _SEMS,
    )


BS = 16
BN = BS * FAN1


def _matT(a, w):
    return lax.dot_general(a, w, (((1,), (1,)), ((), ())),
                           preferred_element_type=jnp.float32)


def _tc_gat_body(h2_ref, hs_ref, h0_ref, wq1, wk1, wv1, ws1,
                 wq2, wk2, wv2, ws2, out_ref):
    f32 = jnp.float32
    scale = float(HD) ** (-0.5)
    SS = (lax.broadcasted_iota(jnp.int32, (D, D), 0) // HD
          == lax.broadcasted_iota(jnp.int32, (D, D), 1) // HD).astype(jnp.bfloat16)

    h2 = h2_ref[...].reshape(FAN2 * BN, D)
    hs = hs_ref[...]
    h0 = h0_ref[...]

    def _gat(h_self, h_neigh, fanout, wq, wk, wv, ws, kv_bf16=False,
             fan_major=False):
        n = h_self.shape[0]
        q = _matT(h_self, wq) * scale
        if kv_bf16:
            hn = h_neigh.astype(jnp.bfloat16)
            k = _matT(hn, wk.astype(jnp.bfloat16))
            v = _matT(hn, wv.astype(jnp.bfloat16))
        else:
            k = _matT(h_neigh, wk)
            v = _matT(h_neigh, wv)
        if fan_major:
            k3 = k.reshape(fanout, n, D)
            kq = (k3 * q[None, :, :]).reshape(fanout * n, D)
        else:
            k3 = k.reshape(n, fanout, D)
            kq = (k3 * q[:, None, :]).reshape(n * fanout, D)
        scf = lax.dot_general(kq.astype(jnp.bfloat16), SS,
                              (((1,), (0,)), ((), ())),
                              preferred_element_type=f32)
        if fan_major:
            p = jnp.exp(scf.reshape(fanout, n, D))
            s = jnp.sum(p, axis=0)
            agg = jnp.sum(p * v.reshape(fanout, n, D), axis=0) / s
        else:
            p = jnp.exp(scf.reshape(n, fanout, D))
            s = jnp.sum(p, axis=1)
            agg = jnp.sum(p * v.reshape(n, fanout, D), axis=1) / s
        return _matT(h_self, ws) + agg

    h1 = jnp.maximum(_gat(hs, h2, FAN2, wq1[...], wk1[...], wv1[...], ws1[...],
                          kv_bf16=True, fan_major=True), 0.0)
    out = jnp.maximum(_gat(h0, h1, FAN1, wq2[...], wk2[...], wv2[...], ws2[...],
                           kv_bf16=True), 0.0)
    out_ref[...] = out


def _tc_gat_chunk(seed_off, nseeds, rows2c, rows1,
                  Wq1, Wk1, Wv1, Ws1, Wq2, Wk2, Wv2, Ws2, interpret=False):
    wspec = pl.BlockSpec((D, D), lambda j: (0, 0))
    gridc = nseeds // BS
    off1 = seed_off // BS
    off0 = N1 // BS + seed_off // BS
    return pl.pallas_call(
        _tc_gat_body,
        grid=(gridc,),
        in_specs=[
            pl.BlockSpec((FAN2, BN, D), lambda j: (0, j, 0)),
            pl.BlockSpec((BN, D), lambda j: (j + off1, 0)),
            pl.BlockSpec((BS, D), lambda j: (j + off0, 0)),
            wspec, wspec, wspec, wspec, wspec, wspec, wspec, wspec,
        ],
        out_specs=pl.BlockSpec((BS, D), lambda j: (j, 0)),
        out_shape=jax.ShapeDtypeStruct((nseeds, D), jnp.float32),
        interpret=interpret,
    )(rows2c, rows1, rows1, Wq1, Wk1, Wv1, Ws1, Wq2, Wk2, Wv2, Ws2)


def kernel(seeds, nbr1, nbr2, emb, Wq1, Wk1, Wv1, Ws1, Wq2, Wk2, Wv2, Ws2):
    idx1 = jnp.concatenate([nbr1, seeds]).astype(jnp.int32)
    weights = (Wq1, Wk1, Wv1, Ws1, Wq2, Wk2, Wv2, Ws2)
    rows1 = _sc_self_fn()(emb, idx1)
    outs = []
    seed_off = 0
    for ns in CHUNK_SEEDS:
        npc = ns * FAN1
        n0 = seed_off * FAN1
        idx2c = nbr2[n0:n0 + npc, :].T.reshape(-1).astype(jnp.int32)
        rows2c = _sc_edge_fn(npc * FAN2)(emb, idx2c)
        outs.append(_tc_gat_chunk(seed_off, ns, rows2c.reshape(FAN2, npc, D),
                                  rows1, *weights))
        seed_off += ns
    return jnp.concatenate(outs, axis=0)


# speedup vs baseline: 1.0226x; 1.0044x over previous
import functools

import jax
import jax.numpy as jnp
from jax import lax
from jax.experimental import pallas as pl
from jax.experimental.pallas import tpu as pltpu
from jax.experimental.pallas import tpu_sc as plsc

D = 128
HEADS = 8
HD = D // HEADS
B = 2048
FAN1 = 16
FAN2 = 16

N1 = B * FAN1
E2 = N1 * FAN2
CHUNK_SEEDS = (256,) * 8
NCHK = len(CHUNK_SEEDS)


_NC = 2
_NS = 16
_NW = _NC * _NS

_CH2 = 128

_R1 = N1 + B
_RPW1 = _R1 // _NW
_CH1 = 64
_NCH1 = _RPW1 // _CH1


_NB = 4


def _gather_loop(tab_hbm, idx_all, out_hbm, base, ch, nch, bufs, gsems, wsems):
    def _gfire(g, s):
        off = pl.multiple_of(g * ch, 8)
        pltpu.make_async_copy(
            tab_hbm.at[idx_all.at[pl.ds(off, ch)]], bufs[s], gsems[s]).start()

    def _gwait(g, s):
        off = pl.multiple_of(g * ch, 8)
        pltpu.make_async_copy(
            tab_hbm.at[idx_all.at[pl.ds(off, ch)]], bufs[s], gsems[s]).wait()

    def _wfire(g, s):
        off = pl.multiple_of(base + g * ch, 8)
        pltpu.make_async_copy(
            bufs[s], out_hbm.at[pl.ds(off, ch)], wsems[s]).start()

    def _wwait(s):
        pltpu.make_async_copy(
            bufs[s], out_hbm.at[pl.ds(base, ch)], wsems[s]).wait()

    ng, rem = nch // _NB, nch % _NB

    def _group(t, carry):
        for s in range(_NB):
            @pl.when(t > 0)
            def _():
                _wwait(s)
            _gfire(_NB * t + s, s)
        for s in range(_NB):
            _gwait(_NB * t + s, s)
            _wfire(_NB * t + s, s)
        return carry

    lax.fori_loop(0, ng, _group, 0)
    for s in range(rem):
        if ng > 0:
            _wwait(s)
        _gfire(ng * _NB + s, s)
    for s in range(rem):
        _gwait(ng * _NB + s, s)
        _wfire(ng * _NB + s, s)
    for s in range(_NB if ng > 0 else rem):
        _wwait(s)


_SEMS = [pltpu.SemaphoreType.DMA] * (2 * _NB)


@functools.cache
def _sc_edge_fn(ec):
    rpw = ec // _NW
    nch = rpw // _CH2

    def body(emb_hbm, idx2_hbm, out2_hbm, idx2_all, *rest):
        bufs, sems = rest[:_NB], rest[_NB:]
        wid = lax.axis_index("s") * _NC + lax.axis_index("c")
        base2 = pl.multiple_of(wid * rpw, 8)
        pltpu.sync_copy(idx2_hbm.at[pl.ds(base2, rpw)], idx2_all)
        _gather_loop(emb_hbm, idx2_all, out2_hbm, base2, _CH2, nch,
                     bufs, sems[:_NB], sems[_NB:])

    return pl.kernel(
        body,
        out_type=jax.ShapeDtypeStruct((ec, D), jnp.float32),
        mesh=plsc.VectorSubcoreMesh(core_axis_name="c", subcore_axis_name="s"),
        scratch_types=[pltpu.VMEM((rpw,), jnp.int32)]
        + [pltpu.VMEM((_CH2, D), jnp.float32)] * _NB + _SEMS,
    )


def _sc_gather_self_body(emb_hbm, idx1_hbm, out1_hbm, idx1_all, *rest):
    bufs, sems = rest[:_NB], rest[_NB:]
    wid = lax.axis_index("s") * _NC + lax.axis_index("c")
    base1 = pl.multiple_of(wid * _RPW1, 8)
    pltpu.sync_copy(idx1_hbm.at[pl.ds(base1, _RPW1)], idx1_all)
    _gather_loop(emb_hbm, idx1_all, out1_hbm, base1, _CH1, _NCH1,
                 bufs, sems[:_NB], sems[_NB:])


@functools.cache
def _sc_self_fn():
    return pl.kernel(
        _sc_gather_self_body,
        out_type=jax.ShapeDtypeStruct((_R1, D), jnp.float32),
        mesh=plsc.VectorSubcoreMesh(core_axis_name="c", subcore_axis_name="s"),
        scratch_types=[pltpu.VMEM((_RPW1,), jnp.int32)]
        + [pltpu.VMEM((_CH1, D), jnp.float32)] * _NB + _SEMS,
    )


BS = 16
BN = BS * FAN1


def _matT(a, w):
    return lax.dot_general(a, w, (((1,), (1,)), ((), ())),
                           preferred_element_type=jnp.float32)


def _tc_gat_body(h2_ref, hs_ref, h0_ref, wq1, wk1, wv1, ws1,
                 wq2, wk2, wv2, ws2, out_ref):
    f32 = jnp.float32
    scale = float(HD) ** (-0.5)
    SS = (lax.broadcasted_iota(jnp.int32, (D, D), 0) // HD
          == lax.broadcasted_iota(jnp.int32, (D, D), 1) // HD).astype(jnp.bfloat16)

    h2 = h2_ref[...].reshape(FAN2 * BN, D)
    hs = hs_ref[...]
    h0 = h0_ref[...]

    def _gat(h_self, h_neigh, fanout, wq, wk, wv, ws, kv_bf16=False,
             fan_major=False):
        n = h_self.shape[0]
        q = _matT(h_self, wq) * scale
        if kv_bf16:
            hn = h_neigh.astype(jnp.bfloat16)
            k = _matT(hn, wk.astype(jnp.bfloat16))
            v = _matT(hn, wv.astype(jnp.bfloat16))
        else:
            k = _matT(h_neigh, wk)
            v = _matT(h_neigh, wv)
        if fan_major:
            k3 = k.reshape(fanout, n, D)
            kq = (k3 * q[None, :, :]).reshape(fanout * n, D)
        else:
            k3 = k.reshape(n, fanout, D)
            kq = (k3 * q[:, None, :]).reshape(n * fanout, D)
        scf = lax.dot_general(kq.astype(jnp.bfloat16), SS,
                              (((1,), (0,)), ((), ())),
                              preferred_element_type=f32)
        if fan_major:
            p = jnp.exp(scf.reshape(fanout, n, D))
            s = jnp.sum(p, axis=0)
            agg = jnp.sum(p * v.reshape(fanout, n, D), axis=0) / s
        else:
            p = jnp.exp(scf.reshape(n, fanout, D))
            s = jnp.sum(p, axis=1)
            agg = jnp.sum(p * v.reshape(n, fanout, D), axis=1) / s
        return _matT(h_self, ws) + agg

    h1 = jnp.maximum(_gat(hs, h2, FAN2, wq1[...], wk1[...], wv1[...], ws1[...],
                          kv_bf16=True, fan_major=True), 0.0)
    out = jnp.maximum(_gat(h0, h1, FAN1, wq2[...], wk2[...], wv2[...], ws2[...],
                           kv_bf16=True), 0.0)
    out_ref[...] = out


def _tc_gat_chunk(seed_off, nseeds, rows2c, rows1,
                  Wq1, Wk1, Wv1, Ws1, Wq2, Wk2, Wv2, Ws2, interpret=False):
    wspec = pl.BlockSpec((D, D), lambda j: (0, 0))
    gridc = nseeds // BS
    off1 = seed_off // BS
    off0 = N1 // BS + seed_off // BS
    return pl.pallas_call(
        _tc_gat_body,
        grid=(gridc,),
        in_specs=[
            pl.BlockSpec((FAN2, BN, D), lambda j: (0, j, 0)),
            pl.BlockSpec((BN, D), lambda j: (j + off1, 0)),
            pl.BlockSpec((BS, D), lambda j: (j + off0, 0)),
            wspec, wspec, wspec, wspec, wspec, wspec, wspec, wspec,
        ],
        out_specs=pl.BlockSpec((BS, D), lambda j: (j, 0)),
        out_shape=jax.ShapeDtypeStruct((nseeds, D), jnp.float32),
        interpret=interpret,
    )(rows2c, rows1, rows1, Wq1, Wk1, Wv1, Ws1, Wq2, Wk2, Wv2, Ws2)


def kernel(seeds, nbr1, nbr2, emb, Wq1, Wk1, Wv1, Ws1, Wq2, Wk2, Wv2, Ws2):
    idx1 = jnp.concatenate([nbr1, seeds]).astype(jnp.int32)
    weights = (Wq1, Wk1, Wv1, Ws1, Wq2, Wk2, Wv2, Ws2)
    rows1 = _sc_self_fn()(emb, idx1)
    outs = []
    seed_off = 0
    for ns in CHUNK_SEEDS:
        npc = ns * FAN1
        n0 = seed_off * FAN1
        idx2c = nbr2[n0:n0 + npc, :].T.reshape(-1).astype(jnp.int32)
        rows2c = _sc_edge_fn(npc * FAN2)(emb, idx2c)
        outs.append(_tc_gat_chunk(seed_off, ns, rows2c.reshape(FAN2, npc, D),
                                  rows1, *weights))
        seed_off += ns
    return jnp.concatenate(outs, axis=0)


# speedup vs baseline: 1.0386x; 1.0156x over previous
import functools

import jax
import jax.numpy as jnp
from jax import lax
from jax.experimental import pallas as pl
from jax.experimental.pallas import tpu as pltpu
from jax.experimental.pallas import tpu_sc as plsc

D = 128
HEADS = 8
HD = D // HEADS
B = 2048
FAN1 = 16
FAN2 = 16

N1 = B * FAN1
E2 = N1 * FAN2
CHUNK_SEEDS = (512, 512, 512, 512)
NCHK = len(CHUNK_SEEDS)


_NC = 2
_NS = 16
_NW = _NC * _NS

_CH2 = 128

_R1 = N1 + B
_RPW1 = _R1 // _NW
_CH1 = 64
_NCH1 = _RPW1 // _CH1


def _gather_loop(tab_hbm, idx_all, out_hbm, base, ch, nch, rows0, rows1, sem0, sem1):
    def _fire(g, rows, sem):
        off = pl.multiple_of(g * ch, 8)
        pltpu.make_async_copy(
            tab_hbm.at[idx_all.at[pl.ds(off, ch)]], rows, sem).start()

    def _drain(g, rows, sem):
        off = pl.multiple_of(g * ch, 8)
        pltpu.make_async_copy(
            tab_hbm.at[idx_all.at[pl.ds(off, ch)]], rows, sem).wait()
        pltpu.sync_copy(rows, out_hbm.at[pl.ds(pl.multiple_of(base + off, 8), ch)])

    _fire(0, rows0, sem0)

    def _pair(t, carry):
        a = 2 * t
        b = a + 1
        _fire(b, rows1, sem1)
        _drain(a, rows0, sem0)

        @pl.when(b + 1 < nch)
        def _():
            _fire(b + 1, rows0, sem0)

        _drain(b, rows1, sem1)
        return carry

    lax.fori_loop(0, nch // 2, _pair, 0)
    if nch % 2:
        _drain(nch - 1, rows0, sem0)


_SEMS = [pltpu.SemaphoreType.DMA, pltpu.SemaphoreType.DMA]


@functools.cache
def _sc_edge_fn(ec):
    rpw = ec // _NW
    nch = rpw // _CH2

    def body(emb_hbm, idx2_hbm, out2_hbm, idx2_all, r2a, r2b, sem0, sem1):
        wid = lax.axis_index("s") * _NC + lax.axis_index("c")
        base2 = pl.multiple_of(wid * rpw, 8)
        pltpu.sync_copy(idx2_hbm.at[pl.ds(base2, rpw)], idx2_all)
        _gather_loop(emb_hbm, idx2_all, out2_hbm, base2, _CH2, nch,
                     r2a, r2b, sem0, sem1)

    return pl.kernel(
        body,
        out_type=jax.ShapeDtypeStruct((ec, D), jnp.float32),
        mesh=plsc.VectorSubcoreMesh(core_axis_name="c", subcore_axis_name="s"),
        scratch_types=[
            pltpu.VMEM((rpw,), jnp.int32),
            pltpu.VMEM((_CH2, D), jnp.float32),
            pltpu.VMEM((_CH2, D), jnp.float32),
        ] + _SEMS,
    )


@functools.cache
def _sc_both_fn(ec):
    rpw = ec // _NW
    nch = rpw // _CH2

    def body(emb_hbm, idx2_hbm, idx1_hbm, out2_hbm, out1_hbm,
             idx2_all, r2a, r2b, idx1_all, r1a, r1b, sem0, sem1):
        wid = lax.axis_index("s") * _NC + lax.axis_index("c")
        base1 = pl.multiple_of(wid * _RPW1, 8)
        pltpu.sync_copy(idx1_hbm.at[pl.ds(base1, _RPW1)], idx1_all)
        _gather_loop(emb_hbm, idx1_all, out1_hbm, base1, _CH1, _NCH1,
                     r1a, r1b, sem0, sem1)
        base2 = pl.multiple_of(wid * rpw, 8)
        pltpu.sync_copy(idx2_hbm.at[pl.ds(base2, rpw)], idx2_all)
        _gather_loop(emb_hbm, idx2_all, out2_hbm, base2, _CH2, nch,
                     r2a, r2b, sem0, sem1)

    return pl.kernel(
        body,
        out_type=(jax.ShapeDtypeStruct((ec, D), jnp.float32),
                  jax.ShapeDtypeStruct((_R1, D), jnp.float32)),
        mesh=plsc.VectorSubcoreMesh(core_axis_name="c", subcore_axis_name="s"),
        scratch_types=[
            pltpu.VMEM((rpw,), jnp.int32),
            pltpu.VMEM((_CH2, D), jnp.float32),
            pltpu.VMEM((_CH2, D), jnp.float32),
            pltpu.VMEM((_RPW1,), jnp.int32),
            pltpu.VMEM((_CH1, D), jnp.float32),
            pltpu.VMEM((_CH1, D), jnp.float32),
        ] + _SEMS,
    )


BS = 16
BN = BS * FAN1


def _matT(a, w):
    return lax.dot_general(a, w, (((1,), (1,)), ((), ())),
                           preferred_element_type=jnp.float32)


def _tc_gat_body(h2_ref, hs_ref, h0_ref, wq1, wk1, wv1, ws1,
                 wq2, wk2, wv2, ws2, out_ref):
    f32 = jnp.float32
    scale = float(HD) ** (-0.5)
    SS = (lax.broadcasted_iota(jnp.int32, (D, D), 0) // HD
          == lax.broadcasted_iota(jnp.int32, (D, D), 1) // HD).astype(jnp.bfloat16)

    h2 = h2_ref[...].reshape(FAN2 * BN, D)
    hs = hs_ref[...]
    h0 = h0_ref[...]

    def _gat(h_self, h_neigh, fanout, wq, wk, wv, ws, kv_bf16=False,
             fan_major=False):
        n = h_self.shape[0]
        q = _matT(h_self, wq) * scale
        if kv_bf16:
            hn = h_neigh.astype(jnp.bfloat16)
            k = _matT(hn, wk.astype(jnp.bfloat16))
            v = _matT(hn, wv.astype(jnp.bfloat16))
        else:
            k = _matT(h_neigh, wk)
            v = _matT(h_neigh, wv)
        if fan_major:
            k3 = k.reshape(fanout, n, D)
            kq = (k3 * q[None, :, :]).reshape(fanout * n, D)
        else:
            k3 = k.reshape(n, fanout, D)
            kq = (k3 * q[:, None, :]).reshape(n * fanout, D)
        scf = lax.dot_general(kq.astype(jnp.bfloat16), SS,
                              (((1,), (0,)), ((), ())),
                              preferred_element_type=f32)
        if fan_major:
            p = jnp.exp(scf.reshape(fanout, n, D))
            s = jnp.sum(p, axis=0)
            agg = jnp.sum(p * v.reshape(fanout, n, D), axis=0) / s
        else:
            p = jnp.exp(scf.reshape(n, fanout, D))
            s = jnp.sum(p, axis=1)
            agg = jnp.sum(p * v.reshape(n, fanout, D), axis=1) / s
        return _matT(h_self, ws) + agg

    h1 = jnp.maximum(_gat(hs, h2, FAN2, wq1[...], wk1[...], wv1[...], ws1[...],
                          kv_bf16=True, fan_major=True), 0.0)
    out = jnp.maximum(_gat(h0, h1, FAN1, wq2[...], wk2[...], wv2[...], ws2[...],
                           kv_bf16=True), 0.0)
    out_ref[...] = out


def _tc_gat_chunk(seed_off, nseeds, rows2c, rows1,
                  Wq1, Wk1, Wv1, Ws1, Wq2, Wk2, Wv2, Ws2, interpret=False):
    wspec = pl.BlockSpec((D, D), lambda j: (0, 0))
    gridc = nseeds // BS
    off1 = seed_off // BS
    off0 = N1 // BS + seed_off // BS
    return pl.pallas_call(
        _tc_gat_body,
        grid=(gridc,),
        in_specs=[
            pl.BlockSpec((FAN2, BN, D), lambda j: (0, j, 0)),
            pl.BlockSpec((BN, D), lambda j: (j + off1, 0)),
            pl.BlockSpec((BS, D), lambda j: (j + off0, 0)),
            wspec, wspec, wspec, wspec, wspec, wspec, wspec, wspec,
        ],
        out_specs=pl.BlockSpec((BS, D), lambda j: (j, 0)),
        out_shape=jax.ShapeDtypeStruct((nseeds, D), jnp.float32),
        interpret=interpret,
    )(rows2c, rows1, rows1, Wq1, Wk1, Wv1, Ws1, Wq2, Wk2, Wv2, Ws2)


def kernel(seeds, nbr1, nbr2, emb, Wq1, Wk1, Wv1, Ws1, Wq2, Wk2, Wv2, Ws2):
    idx1 = jnp.concatenate([nbr1, seeds]).astype(jnp.int32)
    weights = (Wq1, Wk1, Wv1, Ws1, Wq2, Wk2, Wv2, Ws2)
    outs = []
    rows1 = None
    seed_off = 0
    for ns in CHUNK_SEEDS:
        npc = ns * FAN1
        n0 = seed_off * FAN1
        idx2c = nbr2[n0:n0 + npc, :].T.reshape(-1).astype(jnp.int32)
        if rows1 is None:
            rows2c, rows1 = _sc_both_fn(npc * FAN2)(emb, idx2c, idx1)
        else:
            rows2c = _sc_edge_fn(npc * FAN2)(emb, idx2c)
        outs.append(_tc_gat_chunk(seed_off, ns, rows2c.reshape(FAN2, npc, D),
                                  rows1, *weights))
        seed_off += ns
    return jnp.concatenate(outs, axis=0)
